# baseline (speedup 1.0000x reference)
import functools

import jax
import jax.numpy as jnp
from jax import lax
from jax.experimental import pallas as pl
from jax.experimental.pallas import tpu as pltpu
from jax.experimental.pallas import tpu_sc as plsc

_NUM_CORES = 2
_NUM_SUBCORES = 16
_NUM_WORKERS = _NUM_CORES * _NUM_SUBCORES
_LANES = 16


def _embedding_lookup_t(xt, wt):
    s_len, b_len = xt.shape
    d_len, v_len = wt.shape
    d_per_w = d_len // _NUM_WORKERS
    n_gather = b_len // _LANES
    mesh = plsc.VectorSubcoreMesh(core_axis_name="c", subcore_axis_name="s")

    @functools.partial(
        pl.kernel,
        mesh=mesh,
        out_type=jax.ShapeDtypeStruct((s_len, d_len, b_len), jnp.float32),
        scratch_types=[
            pltpu.VMEM((v_len,), jnp.float32),
            pltpu.VMEM((2, b_len), jnp.int32),
            pltpu.VMEM((2, b_len), jnp.float32),
            pltpu.SemaphoreType.DMA((2,)),
            pltpu.SemaphoreType.DMA((2,)),
        ],
        compiler_params=pltpu.CompilerParams(
            use_tc_tiling_on_sc=True, needs_layout_passes=False
        ),
    )
    def k(xt_hbm, wt_hbm, out_hbm, wrow_v, idx_v, outbuf_v, isem, osem):
        wid = lax.axis_index("s") * _NUM_CORES + lax.axis_index("c")
        for p in range(d_per_w):
            d = wid * d_per_w + p
            pltpu.sync_copy(wt_hbm.at[d], wrow_v)
            pltpu.make_async_copy(xt_hbm.at[0], idx_v.at[0], isem.at[0]).start()

            def body(s, carry):
                b = lax.rem(s, 2)

                @pl.when(s + 1 < s_len)
                def _prefetch():
                    nb = lax.rem(s + 1, 2)
                    pltpu.make_async_copy(
                        xt_hbm.at[s + 1], idx_v.at[nb], isem.at[nb]
                    ).start()

                pltpu.make_async_copy(
                    xt_hbm.at[s], idx_v.at[b], isem.at[b]
                ).wait()

                @pl.when(s >= 2)
                def _drain():
                    pltpu.make_async_copy(
                        outbuf_v.at[b], out_hbm.at[s - 2, d], osem.at[b]
                    ).wait()

                def g(i, c):
                    ids = idx_v[b, pl.ds(i * _LANES, _LANES)]
                    outbuf_v[b, pl.ds(i * _LANES, _LANES)] = plsc.load_gather(
                        wrow_v, [ids]
                    )
                    return c

                lax.fori_loop(0, n_gather, g, 0, unroll=8)

                pltpu.make_async_copy(
                    outbuf_v.at[b], out_hbm.at[s, d], osem.at[b]
                ).start()
                return carry

            lax.fori_loop(0, s_len, body, 0)
            for t in (s_len - 2, s_len - 1):
                pltpu.make_async_copy(
                    outbuf_v.at[t % 2], out_hbm.at[t, d], osem.at[t % 2]
                ).wait()

    return k(xt, wt)


def kernel(x, weight):
    b, s = x.shape
    out_t = _embedding_lookup_t(x.T.astype(jnp.int32), weight.T)
    return jnp.transpose(out_t, (2, 0, 1))


# speedup vs baseline: 2.3688x; 2.3688x over previous
import functools

import jax
import jax.numpy as jnp
from jax import lax
from jax.experimental import pallas as pl
from jax.experimental.pallas import tpu as pltpu
from jax.experimental.pallas import tpu_sc as plsc

_NUM_CORES = 2
_NUM_SUBCORES = 16
_NUM_WORKERS = _NUM_CORES * _NUM_SUBCORES
_LANES = 16


def _embedding_lookup_t(xt, wt):
    s_len, b_len = xt.shape
    d_len, v_len = wt.shape
    d_per_w = d_len // _NUM_WORKERS
    n_gather = b_len // _LANES
    mesh = plsc.VectorSubcoreMesh(core_axis_name="c", subcore_axis_name="s")

    @functools.partial(
        pl.kernel,
        mesh=mesh,
        out_type=jax.ShapeDtypeStruct((s_len, d_len, b_len), jnp.float32),
        scratch_types=[
            pltpu.VMEM((v_len,), jnp.float32),
            pltpu.VMEM((2, b_len), jnp.int32),
            pltpu.VMEM((2, b_len), jnp.float32),
            pltpu.SemaphoreType.DMA((2,)),
            pltpu.SemaphoreType.DMA((2,)),
        ],
        compiler_params=pltpu.CompilerParams(
            use_tc_tiling_on_sc=True, needs_layout_passes=False
        ),
    )
    def k(xt_hbm, wt_hbm, out_hbm, wrow_v, idx_v, outbuf_v, isem, osem):
        wid = lax.axis_index("s") * _NUM_CORES + lax.axis_index("c")
        for p in range(d_per_w):
            d = wid * d_per_w + p
            pltpu.sync_copy(wt_hbm.at[d], wrow_v)
            pltpu.make_async_copy(xt_hbm.at[0], idx_v.at[0], isem.at[0]).start()

            def body(s, carry):
                b = lax.rem(s, 2)

                @pl.when(s + 1 < s_len)
                def _prefetch():
                    nb = lax.rem(s + 1, 2)
                    pltpu.make_async_copy(
                        xt_hbm.at[s + 1], idx_v.at[nb], isem.at[nb]
                    ).start()

                pltpu.make_async_copy(
                    xt_hbm.at[s], idx_v.at[b], isem.at[b]
                ).wait()

                @pl.when(s >= 2)
                def _drain():
                    pltpu.make_async_copy(
                        outbuf_v.at[b], out_hbm.at[s - 2, d], osem.at[b]
                    ).wait()

                @plsc.parallel_loop(0, b_len, step=_LANES, unroll=8)
                def _gather(i):
                    ids = idx_v[b, pl.ds(i, _LANES)]
                    outbuf_v[b, pl.ds(i, _LANES)] = plsc.load_gather(
                        wrow_v, [ids]
                    )

                pltpu.make_async_copy(
                    outbuf_v.at[b], out_hbm.at[s, d], osem.at[b]
                ).start()
                return carry

            lax.fori_loop(0, s_len, body, 0)
            for t in (s_len - 2, s_len - 1):
                pltpu.make_async_copy(
                    outbuf_v.at[t % 2], out_hbm.at[t, d], osem.at[t % 2]
                ).wait()

    return k(xt, wt)


def kernel(x, weight):
    b, s = x.shape
    out_t = _embedding_lookup_t(x.T.astype(jnp.int32), weight.T)
    return jnp.transpose(out_t, (2, 0, 1))


# speedup vs baseline: 2.3850x; 1.0068x over previous
import functools

import jax
import jax.numpy as jnp
from jax import lax
from jax.experimental import pallas as pl
from jax.experimental.pallas import tpu as pltpu
from jax.experimental.pallas import tpu_sc as plsc

_NUM_CORES = 2
_NUM_SUBCORES = 16
_NUM_WORKERS = _NUM_CORES * _NUM_SUBCORES
_LANES = 16


def _embedding_lookup_t(xt, wt):
    s_len, b_len = xt.shape
    d_len, v_len = wt.shape
    d_per_w = d_len // _NUM_WORKERS
    n_gather = b_len // _LANES
    mesh = plsc.VectorSubcoreMesh(core_axis_name="c", subcore_axis_name="s")

    @functools.partial(
        pl.kernel,
        mesh=mesh,
        out_type=jax.ShapeDtypeStruct((s_len, d_len, b_len), jnp.float32),
        scratch_types=[
            pltpu.VMEM((v_len,), jnp.float32),
            pltpu.VMEM((2, b_len), jnp.int32),
            pltpu.VMEM((2, b_len), jnp.float32),
            pltpu.SemaphoreType.DMA((2,)),
            pltpu.SemaphoreType.DMA((2,)),
        ],
        compiler_params=pltpu.CompilerParams(
            use_tc_tiling_on_sc=True, needs_layout_passes=False
        ),
    )
    def k(xt_hbm, wt_hbm, out_hbm, wrow_v, idx_v, outbuf_v, isem, osem):
        wid = lax.axis_index("s") * _NUM_CORES + lax.axis_index("c")
        for p in range(d_per_w):
            d = wid * d_per_w + p
            pltpu.sync_copy(wt_hbm.at[d], wrow_v)
            pltpu.make_async_copy(xt_hbm.at[0], idx_v.at[0], isem.at[0]).start()

            def body(s, carry):
                b = lax.rem(s, 2)

                @pl.when(s + 1 < s_len)
                def _prefetch():
                    nb = lax.rem(s + 1, 2)
                    pltpu.make_async_copy(
                        xt_hbm.at[s + 1], idx_v.at[nb], isem.at[nb]
                    ).start()

                pltpu.make_async_copy(
                    xt_hbm.at[s], idx_v.at[b], isem.at[b]
                ).wait()

                @pl.when(s >= 2)
                def _drain():
                    pltpu.make_async_copy(
                        outbuf_v.at[b], out_hbm.at[s - 2, d], osem.at[b]
                    ).wait()

                @plsc.parallel_loop(0, b_len, step=_LANES, unroll=16)
                def _gather(i):
                    ids = idx_v[b, pl.ds(i, _LANES)]
                    outbuf_v[b, pl.ds(i, _LANES)] = plsc.load_gather(
                        wrow_v, [ids]
                    )

                pltpu.make_async_copy(
                    outbuf_v.at[b], out_hbm.at[s, d], osem.at[b]
                ).start()
                return carry

            lax.fori_loop(0, s_len, body, 0)
            for t in (s_len - 2, s_len - 1):
                pltpu.make_async_copy(
                    outbuf_v.at[t % 2], out_hbm.at[t, d], osem.at[t % 2]
                ).wait()

    return k(xt, wt)


def kernel(x, weight):
    b, s = x.shape
    out_t = _embedding_lookup_t(x.T.astype(jnp.int32), weight.T)
    return jnp.transpose(out_t, (2, 0, 1))


# speedup vs baseline: 3.2248x; 1.3521x over previous
import functools

import jax
import jax.numpy as jnp
from jax import lax
from jax.experimental import pallas as pl
from jax.experimental.pallas import tpu as pltpu
from jax.experimental.pallas import tpu_sc as plsc

_NUM_CORES = 2
_NUM_SUBCORES = 16
_NUM_WORKERS = _NUM_CORES * _NUM_SUBCORES
_LANES = 16


def _embedding_lookup_t(xt, wt):
    s_len, b_len = xt.shape
    d_len, v_len = wt.shape
    d_per_w = d_len // _NUM_WORKERS
    n_gather = b_len // _LANES
    mesh = plsc.VectorSubcoreMesh(core_axis_name="c", subcore_axis_name="s")

    @functools.partial(
        pl.kernel,
        mesh=mesh,
        out_type=jax.ShapeDtypeStruct((s_len, d_len, b_len), jnp.float32),
        scratch_types=[
            pltpu.VMEM((v_len,), jnp.float32),
            pltpu.VMEM((2, b_len), jnp.int32),
            pltpu.VMEM((2, b_len), jnp.float32),
            pltpu.VMEM_SHARED((s_len * b_len,), jnp.int32),
            pltpu.SemaphoreType.DMA((2,)),
            pltpu.SemaphoreType.DMA((2,)),
        ],
        compiler_params=pltpu.CompilerParams(
            use_tc_tiling_on_sc=True, needs_layout_passes=False
        ),
    )
    def k(xt_hbm, wt_hbm, out_hbm, wrow_v, idx_v, outbuf_v, xt_sp, isem, osem):
        ss = lax.axis_index("s")
        wid = ss * _NUM_CORES + lax.axis_index("c")

        for r in range((s_len + _NUM_SUBCORES - 1) // _NUM_SUBCORES):
            s_row = ss + _NUM_SUBCORES * r

            @pl.when(s_row < s_len)
            def _stage():
                pltpu.sync_copy(
                    xt_hbm.at[s_row], xt_sp.at[pl.ds(s_row * b_len, b_len)]
                )

        plsc.subcore_barrier()

        for p in range(d_per_w):
            d = wid * d_per_w + p
            pltpu.sync_copy(wt_hbm.at[d], wrow_v)
            pltpu.make_async_copy(
                xt_sp.at[pl.ds(0, b_len)], idx_v.at[0], isem.at[0]
            ).start()

            def body(s, carry):
                b = lax.rem(s, 2)

                @pl.when(s + 1 < s_len)
                def _prefetch():
                    nb = lax.rem(s + 1, 2)
                    pltpu.make_async_copy(
                        xt_sp.at[pl.ds((s + 1) * b_len, b_len)],
                        idx_v.at[nb],
                        isem.at[nb],
                    ).start()

                pltpu.make_async_copy(
                    xt_sp.at[pl.ds(s * b_len, b_len)], idx_v.at[b], isem.at[b]
                ).wait()

                @pl.when(s >= 2)
                def _drain():
                    pltpu.make_async_copy(
                        outbuf_v.at[b], out_hbm.at[s - 2, d], osem.at[b]
                    ).wait()

                @plsc.parallel_loop(0, b_len, step=_LANES, unroll=16)
                def _gather(i):
                    ids = idx_v[b, pl.ds(i, _LANES)]
                    outbuf_v[b, pl.ds(i, _LANES)] = plsc.load_gather(
                        wrow_v, [ids]
                    )

                pltpu.make_async_copy(
                    outbuf_v.at[b], out_hbm.at[s, d], osem.at[b]
                ).start()
                return carry

            lax.fori_loop(0, s_len, body, 0)
            for t in (s_len - 2, s_len - 1):
                pltpu.make_async_copy(
                    outbuf_v.at[t % 2], out_hbm.at[t, d], osem.at[t % 2]
                ).wait()

    return k(xt, wt)


def kernel(x, weight):
    b, s = x.shape
    out_t = _embedding_lookup_t(x.T.astype(jnp.int32), weight.T)
    return jnp.transpose(out_t, (2, 0, 1))


# speedup vs baseline: 3.3420x; 1.0364x over previous
import functools

import jax
import jax.numpy as jnp
from jax import lax
from jax.experimental import pallas as pl
from jax.experimental.pallas import tpu as pltpu
from jax.experimental.pallas import tpu_sc as plsc

_NUM_CORES = 2
_NUM_SUBCORES = 16
_NUM_WORKERS = _NUM_CORES * _NUM_SUBCORES
_LANES = 16


def _embedding_lookup_t(xt, wt):
    s_len, b_len = xt.shape
    d_len, v_len = wt.shape
    d_per_w = d_len // _NUM_WORKERS
    n_gather = b_len // _LANES
    mesh = plsc.VectorSubcoreMesh(core_axis_name="c", subcore_axis_name="s")

    @functools.partial(
        pl.kernel,
        mesh=mesh,
        out_type=jax.ShapeDtypeStruct((s_len, d_len, b_len), jnp.float32),
        scratch_types=[
            pltpu.VMEM((v_len,), jnp.float32),
            pltpu.VMEM((2, b_len), jnp.int32),
            pltpu.VMEM((2, b_len), jnp.float32),
            pltpu.VMEM_SHARED((s_len * b_len,), jnp.int32),
            pltpu.SemaphoreType.DMA((2,)),
            pltpu.SemaphoreType.DMA((2,)),
            pltpu.SemaphoreType.DMA,
        ],
        compiler_params=pltpu.CompilerParams(
            use_tc_tiling_on_sc=True, needs_layout_passes=False
        ),
    )
    def k(
        xt_hbm, wt_hbm, out_hbm, wrow_v, idx_v, outbuf_v, xt_sp, isem, osem, wsem
    ):
        ss = lax.axis_index("s")
        wid = ss * _NUM_CORES + lax.axis_index("c")

        wt_first = pltpu.make_async_copy(
            wt_hbm.at[wid * d_per_w], wrow_v, wsem
        )
        wt_first.start()

        for r in range((s_len + _NUM_SUBCORES - 1) // _NUM_SUBCORES):
            s_row = ss + _NUM_SUBCORES * r

            @pl.when(s_row < s_len)
            def _stage():
                pltpu.sync_copy(
                    xt_hbm.at[s_row], xt_sp.at[pl.ds(s_row * b_len, b_len)]
                )

        plsc.subcore_barrier()

        for p in range(d_per_w):
            d = wid * d_per_w + p
            if p == 0:
                wt_first.wait()
            else:
                pltpu.sync_copy(wt_hbm.at[d], wrow_v)
            pltpu.make_async_copy(
                xt_sp.at[pl.ds(0, b_len)], idx_v.at[0], isem.at[0]
            ).start()

            def body(s, carry):
                b = lax.rem(s, 2)

                @pl.when(s + 1 < s_len)
                def _prefetch():
                    nb = lax.rem(s + 1, 2)
                    pltpu.make_async_copy(
                        xt_sp.at[pl.ds((s + 1) * b_len, b_len)],
                        idx_v.at[nb],
                        isem.at[nb],
                    ).start()

                pltpu.make_async_copy(
                    xt_sp.at[pl.ds(s * b_len, b_len)], idx_v.at[b], isem.at[b]
                ).wait()

                @pl.when(s >= 2)
                def _drain():
                    pltpu.make_async_copy(
                        outbuf_v.at[b], out_hbm.at[s - 2, d], osem.at[b]
                    ).wait()

                @plsc.parallel_loop(0, b_len, step=_LANES, unroll=16)
                def _gather(i):
                    ids = idx_v[b, pl.ds(i, _LANES)]
                    outbuf_v[b, pl.ds(i, _LANES)] = plsc.load_gather(
                        wrow_v, [ids]
                    )

                pltpu.make_async_copy(
                    outbuf_v.at[b], out_hbm.at[s, d], osem.at[b]
                ).start()
                return carry

            lax.fori_loop(0, s_len, body, 0)
            for t in (s_len - 2, s_len - 1):
                pltpu.make_async_copy(
                    outbuf_v.at[t % 2], out_hbm.at[t, d], osem.at[t % 2]
                ).wait()

    return k(xt, wt)


def kernel(x, weight):
    b, s = x.shape
    out_t = _embedding_lookup_t(x.T.astype(jnp.int32), weight.T)
    return jnp.transpose(out_t, (2, 0, 1))


# speedup vs baseline: 3.3582x; 1.0049x over previous
import functools

import jax
import jax.numpy as jnp
from jax import lax
from jax.experimental import pallas as pl
from jax.experimental.pallas import tpu as pltpu
from jax.experimental.pallas import tpu_sc as plsc

_NUM_CORES = 2
_NUM_SUBCORES = 16
_NUM_WORKERS = _NUM_CORES * _NUM_SUBCORES
_LANES = 16


def _embedding_lookup_t(xt, wt):
    s_len, b_len = xt.shape
    d_len, v_len = wt.shape
    d_per_w = d_len // _NUM_WORKERS
    n_gather = b_len // _LANES
    mesh = plsc.VectorSubcoreMesh(core_axis_name="c", subcore_axis_name="s")

    @functools.partial(
        pl.kernel,
        mesh=mesh,
        out_type=jax.ShapeDtypeStruct((s_len, d_len, b_len), jnp.float32),
        scratch_types=[
            pltpu.VMEM((v_len,), jnp.float32),
            pltpu.VMEM((2, b_len), jnp.int32),
            pltpu.VMEM((2, b_len), jnp.float32),
            pltpu.VMEM_SHARED((s_len * b_len,), jnp.int32),
            pltpu.SemaphoreType.DMA((2,)),
            pltpu.SemaphoreType.DMA((2,)),
            pltpu.SemaphoreType.DMA,
        ],
        compiler_params=pltpu.CompilerParams(
            use_tc_tiling_on_sc=True, needs_layout_passes=False
        ),
    )
    def k(
        xt_hbm, wt_hbm, out_hbm, wrow_v, idx_v, outbuf_v, xt_sp, isem, osem, wsem
    ):
        ss = lax.axis_index("s")
        wid = ss * _NUM_CORES + lax.axis_index("c")

        wt_first = pltpu.make_async_copy(
            wt_hbm.at[wid * d_per_w], wrow_v, wsem
        )
        wt_first.start()

        for r in range((s_len + _NUM_SUBCORES - 1) // _NUM_SUBCORES):
            s_row = ss + _NUM_SUBCORES * r

            @pl.when(s_row < s_len)
            def _stage():
                pltpu.sync_copy(
                    xt_hbm.at[s_row], xt_sp.at[pl.ds(s_row * b_len, b_len)]
                )

        plsc.subcore_barrier()

        for p in range(d_per_w):
            d = wid * d_per_w + p
            if p == 0:
                wt_first.wait()
            else:
                pltpu.sync_copy(wt_hbm.at[d], wrow_v)
            pltpu.make_async_copy(
                xt_sp.at[pl.ds(0, b_len)], idx_v.at[0], isem.at[0]
            ).start()

            def body(s, carry):
                b = lax.rem(s, 2)

                @pl.when(s + 1 < s_len)
                def _prefetch():
                    nb = lax.rem(s + 1, 2)
                    pltpu.make_async_copy(
                        xt_sp.at[pl.ds((s + 1) * b_len, b_len)],
                        idx_v.at[nb],
                        isem.at[nb],
                    ).start()

                pltpu.make_async_copy(
                    xt_sp.at[pl.ds(s * b_len, b_len)], idx_v.at[b], isem.at[b]
                ).wait()

                @pl.when(s >= 2)
                def _drain():
                    pltpu.make_async_copy(
                        outbuf_v.at[b], out_hbm.at[s - 2, d], osem.at[b]
                    ).wait()

                @plsc.parallel_loop(0, b_len, step=_LANES, unroll=32)
                def _gather(i):
                    ids = idx_v[b, pl.ds(i, _LANES)]
                    outbuf_v[b, pl.ds(i, _LANES)] = plsc.load_gather(
                        wrow_v, [ids]
                    )

                pltpu.make_async_copy(
                    outbuf_v.at[b], out_hbm.at[s, d], osem.at[b]
                ).start()
                return carry

            lax.fori_loop(0, s_len, body, 0)
            for t in (s_len - 2, s_len - 1):
                pltpu.make_async_copy(
                    outbuf_v.at[t % 2], out_hbm.at[t, d], osem.at[t % 2]
                ).wait()

    return k(xt, wt)


def kernel(x, weight):
    b, s = x.shape
    out_t = _embedding_lookup_t(x.T.astype(jnp.int32), weight.T)
    return jnp.transpose(out_t, (2, 0, 1))


# speedup vs baseline: 3.3645x; 1.0019x over previous
import functools

import jax
import jax.numpy as jnp
from jax import lax
from jax.experimental import pallas as pl
from jax.experimental.pallas import tpu as pltpu
from jax.experimental.pallas import tpu_sc as plsc

_NUM_CORES = 2
_NUM_SUBCORES = 16
_NUM_WORKERS = _NUM_CORES * _NUM_SUBCORES
_LANES = 16


def _embedding_lookup_t(xt, wt):
    s_len, b_len = xt.shape
    d_len, v_len = wt.shape
    d_per_w = d_len // _NUM_WORKERS
    n_gather = b_len // _LANES
    mesh = plsc.VectorSubcoreMesh(core_axis_name="c", subcore_axis_name="s")

    @functools.partial(
        pl.kernel,
        mesh=mesh,
        out_type=jax.ShapeDtypeStruct((s_len, d_len, b_len), jnp.float32),
        scratch_types=[
            pltpu.VMEM((v_len,), jnp.float32),
            pltpu.VMEM((2, b_len), jnp.int32),
            pltpu.VMEM((2, b_len), jnp.float32),
            pltpu.VMEM_SHARED((s_len * b_len,), jnp.int32),
            pltpu.SemaphoreType.DMA((2,)),
            pltpu.SemaphoreType.DMA((2,)),
            pltpu.SemaphoreType.DMA,
        ],
        compiler_params=pltpu.CompilerParams(
            use_tc_tiling_on_sc=True, needs_layout_passes=False
        ),
    )
    def k(
        xt_hbm, wt_hbm, out_hbm, wrow_v, idx_v, outbuf_v, xt_sp, isem, osem, wsem
    ):
        ss = lax.axis_index("s")
        wid = ss * _NUM_CORES + lax.axis_index("c")

        wt_first = pltpu.make_async_copy(
            wt_hbm.at[wid * d_per_w], wrow_v, wsem
        )
        wt_first.start()

        for r in range((s_len + _NUM_SUBCORES - 1) // _NUM_SUBCORES):
            s_row = ss + _NUM_SUBCORES * r

            @pl.when(s_row < s_len)
            def _stage():
                pltpu.sync_copy(
                    xt_hbm.at[s_row], xt_sp.at[pl.ds(s_row * b_len, b_len)]
                )

        plsc.subcore_barrier()

        wt_pending = wt_first
        for p in range(d_per_w):
            d = wid * d_per_w + p
            wt_pending.wait()
            pltpu.make_async_copy(
                xt_sp.at[pl.ds(0, b_len)], idx_v.at[0], isem.at[0]
            ).start()

            def body(s, carry):
                b = lax.rem(s, 2)

                @pl.when(s + 1 < s_len)
                def _prefetch():
                    nb = lax.rem(s + 1, 2)
                    pltpu.make_async_copy(
                        xt_sp.at[pl.ds((s + 1) * b_len, b_len)],
                        idx_v.at[nb],
                        isem.at[nb],
                    ).start()

                pltpu.make_async_copy(
                    xt_sp.at[pl.ds(s * b_len, b_len)], idx_v.at[b], isem.at[b]
                ).wait()

                @pl.when(s >= 2)
                def _drain():
                    pltpu.make_async_copy(
                        outbuf_v.at[b], out_hbm.at[s - 2, d], osem.at[b]
                    ).wait()

                @plsc.parallel_loop(0, b_len, step=_LANES, unroll=32)
                def _gather(i):
                    ids = idx_v[b, pl.ds(i, _LANES)]
                    outbuf_v[b, pl.ds(i, _LANES)] = plsc.load_gather(
                        wrow_v, [ids]
                    )

                pltpu.make_async_copy(
                    outbuf_v.at[b], out_hbm.at[s, d], osem.at[b]
                ).start()
                return carry

            lax.fori_loop(0, s_len, body, 0)
            if p + 1 < d_per_w:
                wt_pending = pltpu.make_async_copy(
                    wt_hbm.at[d + 1], wrow_v, wsem
                )
                wt_pending.start()
            for t in (s_len - 2, s_len - 1):
                pltpu.make_async_copy(
                    outbuf_v.at[t % 2], out_hbm.at[t, d], osem.at[t % 2]
                ).wait()

    return k(xt, wt)


def kernel(x, weight):
    b, s = x.shape
    out_t = _embedding_lookup_t(x.T.astype(jnp.int32), weight.T)
    return jnp.transpose(out_t, (2, 0, 1))
